# Initial kernel scaffold; baseline (speedup 1.0000x reference)
import jax
import jax.numpy as jnp
from jax.experimental import pallas as pl


def kernel(nodes, edges, senders, receivers, enc_node_W0, enc_node_b0, enc_node_W1, enc_node_b1, enc_edge_W0, enc_edge_b0, enc_edge_W1, enc_edge_b1, W_message, W_node, nodeMLP_W0, nodeMLP_b0, nodeMLP_W1, nodeMLP_b1, ln_scale, ln_bias, dec_W0, dec_b0, dec_W1, dec_b1):
    raise NotImplementedError("write your pallas kernel here")


# speedup vs baseline: 6.4688x; 6.4688x over previous
import functools

import jax
import jax.numpy as jnp
from jax import lax
from jax.experimental import pallas as pl
from jax.experimental.pallas import tpu as pltpu
from jax.experimental.pallas import tpu_sc as plsc

_N = 10000
_E = 320000
_DF = 128
_H = 32
_NMP = 5

_NC = 2
_NS = 16
_NW = _NC * _NS
_CHUNK = 128
_CPT = 80
_EPT = _CPT * _CHUNK
_EPAD = _NW * _EPT
_NP = 10240
_RPT = _NP // _NS

_mesh = plsc.VectorSubcoreMesh(core_axis_name="c", subcore_axis_name="s")
_sc_params = pltpu.CompilerParams(use_tc_tiling_on_sc=False)


def _zero_shared_rows(zbuf, s_sh, sid):
    def zb(i, carry):
        zbuf[i, 0:16] = jnp.zeros((16,), jnp.float32)
        zbuf[i, 16:32] = jnp.zeros((16,), jnp.float32)
        return carry
    lax.fori_loop(0, _CHUNK, zb, 0)
    for k in range(_RPT // _CHUNK):
        pltpu.sync_copy(zbuf, s_sh.at[pl.ds(sid * _RPT + k * _CHUNK, _CHUNK)])


def _drain_shared(s_sh, out, cid, sid):
    pltpu.sync_copy(
        s_sh.at[pl.ds(sid * _RPT, _RPT)],
        out.at[cid, pl.ds(sid * _RPT, _RPT)],
    )


@functools.partial(
    pl.kernel,
    mesh=_mesh,
    out_type=jax.ShapeDtypeStruct((_NC, _NP, _H), jnp.float32),
    scratch_types=[
        pltpu.VMEM((_CPT, _CHUNK), jnp.int32),
        pltpu.VMEM((_CPT, _CHUNK), jnp.int32),
        pltpu.VMEM((_CHUNK, _H), jnp.float32),
        pltpu.VMEM((_CHUNK, _H), jnp.float32),
        pltpu.VMEM_SHARED((_NP, _H), jnp.float32),
        pltpu.SemaphoreType.DMA,
    ],
    compiler_params=_sc_params,
)
def _sc_gather_scatter(hn, send, recv, out, sidx, ridx, gbuf, zbuf, s_sh, sem):
    cid = lax.axis_index("c")
    sid = lax.axis_index("s")
    wid = cid * _NS + sid
    pltpu.sync_copy(send.at[wid], sidx)
    pltpu.sync_copy(recv.at[wid], ridx)
    _zero_shared_rows(zbuf, s_sh, sid)
    plsc.subcore_barrier()

    def step(j, carry):
        pltpu.async_copy(hn.at[sidx.at[j]], gbuf, sem).wait()
        pltpu.sync_copy(gbuf, s_sh.at[ridx.at[j]], add=True)
        return carry
    lax.fori_loop(0, _CPT, step, 0)

    plsc.subcore_barrier()
    _drain_shared(s_sh, out, cid, sid)


@functools.partial(
    pl.kernel,
    mesh=_mesh,
    out_type=jax.ShapeDtypeStruct((_NC, _NP, _H), jnp.float32),
    scratch_types=[
        pltpu.VMEM((_CPT, _CHUNK), jnp.int32),
        pltpu.VMEM((_CHUNK, _H), jnp.float32),
        pltpu.VMEM((_CHUNK, _H), jnp.float32),
        pltpu.VMEM_SHARED((_NP, _H), jnp.float32),
    ],
    compiler_params=_sc_params,
)
def _sc_segment_sum(vals, recv, out, ridx, gbuf, zbuf, s_sh):
    cid = lax.axis_index("c")
    sid = lax.axis_index("s")
    wid = cid * _NS + sid
    pltpu.sync_copy(recv.at[wid], ridx)
    _zero_shared_rows(zbuf, s_sh, sid)
    plsc.subcore_barrier()

    def step(j, carry):
        base = (wid * _CPT + j) * _CHUNK
        pltpu.sync_copy(vals.at[pl.ds(base, _CHUNK)], gbuf)
        pltpu.sync_copy(gbuf, s_sh.at[ridx.at[j]], add=True)
        return carry
    lax.fori_loop(0, _CPT, step, 0)

    plsc.subcore_barrier()
    _drain_shared(s_sh, out, cid, sid)


def _full(shape):
    return pl.BlockSpec(shape, lambda i: (0,) * len(shape))


def _mlp2_kernel(x_ref, w1_ref, b1_ref, w2_ref, b2_ref, o_ref):
    y = jnp.dot(x_ref[...], w1_ref[...], preferred_element_type=jnp.float32)
    y = jnp.maximum(y + b1_ref[...], 0.0)
    o_ref[...] = jnp.dot(y, w2_ref[...], preferred_element_type=jnp.float32) + b2_ref[...]


def _mlp2(x, w1, b1, w2, b2, rows_per_block):
    rows, din = x.shape
    dmid = w1.shape[1]
    dout = w2.shape[1]
    grid = rows // rows_per_block
    return pl.pallas_call(
        _mlp2_kernel,
        grid=(grid,),
        in_specs=[
            pl.BlockSpec((rows_per_block, din), lambda i: (i, 0)),
            _full((din, dmid)), _full((1, dmid)),
            _full((dmid, dout)), _full((1, dout)),
        ],
        out_specs=pl.BlockSpec((rows_per_block, dout), lambda i: (i, 0)),
        out_shape=jax.ShapeDtypeStruct((rows, dout), jnp.float32),
    )(x, w1, b1.reshape(1, -1), w2, b2.reshape(1, -1))


def _update_kernel(hn_ref, s0_ref, s1_ref, agge_ref, wmt_ref, w0a_ref, w0b_ref,
                   b0_ref, w1_ref, b1_ref, wnode_ref, lns_ref, lnb_ref, o_ref):
    h = hn_ref[...]
    s = s0_ref[...] + s1_ref[...]
    agg = jnp.dot(s, wmt_ref[...], preferred_element_type=jnp.float32) + agge_ref[...]
    t = (jnp.dot(h, w0a_ref[...], preferred_element_type=jnp.float32)
         + jnp.dot(agg, w0b_ref[...], preferred_element_type=jnp.float32)
         + b0_ref[...])
    t = jnp.maximum(t, 0.0)
    no = jnp.dot(t, w1_ref[...], preferred_element_type=jnp.float32) + b1_ref[...]
    r = jnp.dot(h, wnode_ref[...], preferred_element_type=jnp.float32) + no
    mu = jnp.mean(r, axis=-1, keepdims=True)
    var = jnp.mean((r - mu) * (r - mu), axis=-1, keepdims=True)
    o_ref[...] = (r - mu) * lax.rsqrt(var + 1e-6) * lns_ref[...] + lnb_ref[...]


def _update(hn, s0, s1, agge, wmt, w0a, w0b, b0, w1, b1, wnode, lns, lnb):
    rb = 1000
    grid = _N // rb
    row = lambda i: (i, 0)
    return pl.pallas_call(
        _update_kernel,
        grid=(grid,),
        in_specs=[
            pl.BlockSpec((rb, _H), row),
            pl.BlockSpec((rb, _H), row),
            pl.BlockSpec((rb, _H), row),
            pl.BlockSpec((rb, _H), row),
            _full((_H, _H)), _full((_H, _H)), _full((_H, _H)), _full((1, _H)),
            _full((_H, _H)), _full((1, _H)), _full((_H, _H)),
            _full((1, _H)), _full((1, _H)),
        ],
        out_specs=pl.BlockSpec((rb, _H), row),
        out_shape=jax.ShapeDtypeStruct((_N, _H), jnp.float32),
    )(hn, s0, s1, agge, wmt, w0a, w0b, b0.reshape(1, -1), w1, b1.reshape(1, -1),
      wnode, lns.reshape(1, -1), lnb.reshape(1, -1))


def kernel(nodes, edges, senders, receivers,
           enc_node_W0, enc_node_b0, enc_node_W1, enc_node_b1,
           enc_edge_W0, enc_edge_b0, enc_edge_W1, enc_edge_b1,
           W_message, W_node,
           nodeMLP_W0, nodeMLP_b0, nodeMLP_W1, nodeMLP_b1,
           ln_scale, ln_bias,
           dec_W0, dec_b0, dec_W1, dec_b1):
    f32 = jnp.float32

    pad = _EPAD - _E
    senders = senders.astype(jnp.int32)
    receivers = receivers.astype(jnp.int32)
    ar = jnp.arange(pad, dtype=jnp.int32)
    pad_send = (ar * 37) % _N
    pad_recv = _N + ar % (_NP - _N)
    send_p = jnp.concatenate([senders, pad_send]).reshape(_NW, _CPT, _CHUNK)
    recv_p = jnp.concatenate([receivers, pad_recv]).reshape(_NW, _CPT, _CHUNK)

    edges_p = jnp.concatenate([edges, jnp.zeros((pad, edges.shape[1]), f32)])
    edges_r = edges_p.reshape(_EPAD // 8, 32)

    wm_top = W_message[:_H]
    wm_bot = W_message[_H:]
    eye8 = jnp.eye(8, dtype=f32)
    w2c = enc_edge_W1 @ wm_bot
    b2c = enc_edge_b1 @ wm_bot
    w1k = jnp.kron(eye8, enc_edge_W0)
    b1k = jnp.tile(enc_edge_b0, 8)
    w2k = jnp.kron(eye8, w2c)
    b2k = jnp.tile(b2c, 8)

    h_n = _mlp2(nodes, enc_node_W0, enc_node_b0, enc_node_W1, enc_node_b1, 1000)
    z_e = _mlp2(edges_r, w1k, b1k, w2k, b2k, 4096).reshape(_EPAD, _H)

    agge_p = _sc_segment_sum(z_e, recv_p)
    agg_e = agge_p[0, :_N] + agge_p[1, :_N]

    w0a = nodeMLP_W0[:_H]
    w0b = nodeMLP_W0[_H:]

    for _ in range(_NMP):
        s_p = _sc_gather_scatter(h_n, send_p, recv_p)
        h_n = _update(h_n, s_p[0, :_N], s_p[1, :_N], agg_e,
                      wm_top, w0a, w0b, nodeMLP_b0, nodeMLP_W1, nodeMLP_b1,
                      W_node, ln_scale, ln_bias)

    return _mlp2(h_n, dec_W0, dec_b0, dec_W1, dec_b1, 1000)


# speedup vs baseline: 9.6359x; 1.4896x over previous
import functools

import jax
import jax.numpy as jnp
from jax import lax
from jax.experimental import pallas as pl
from jax.experimental.pallas import tpu as pltpu
from jax.experimental.pallas import tpu_sc as plsc

_N = 10000
_E = 320000
_DF = 128
_H = 32
_NMP = 5

_NC = 2
_NS = 16
_NW = _NC * _NS
_CHUNK = 128
_CPT = 80
_EPT = _CPT * _CHUNK
_EPAD = _NW * _EPT
_NP = 10240
_RPT = _NP // _NS

_mesh = plsc.VectorSubcoreMesh(core_axis_name="c", subcore_axis_name="s")
_sc_params = pltpu.CompilerParams(use_tc_tiling_on_sc=False)


def _zero_shared_rows(zbuf, s_sh, sid):
    def zb(i, carry):
        zbuf[i, 0:16] = jnp.zeros((16,), jnp.float32)
        zbuf[i, 16:32] = jnp.zeros((16,), jnp.float32)
        return carry
    lax.fori_loop(0, _CHUNK, zb, 0)
    for k in range(_RPT // _CHUNK):
        pltpu.sync_copy(zbuf, s_sh.at[pl.ds(sid * _RPT + k * _CHUNK, _CHUNK)])


def _drain_shared(s_sh, out, cid, sid):
    pltpu.sync_copy(
        s_sh.at[pl.ds(sid * _RPT, _RPT)],
        out.at[cid, pl.ds(sid * _RPT, _RPT)],
    )


@functools.partial(
    pl.kernel,
    mesh=_mesh,
    out_type=jax.ShapeDtypeStruct((_NC, _NP, _H), jnp.float32),
    scratch_types=[
        pltpu.VMEM((_EPT,), jnp.int32),
        pltpu.VMEM((_EPT,), jnp.int32),
        pltpu.VMEM((2, _CHUNK, _H), jnp.float32),
        pltpu.VMEM((_CHUNK, _H), jnp.float32),
        pltpu.VMEM_SHARED((_NP, _H), jnp.float32),
        pltpu.SemaphoreType.DMA,
        pltpu.SemaphoreType.DMA,
    ],
    compiler_params=_sc_params,
)
def _sc_gather_scatter(hn, send, recv, out, sidx, ridx, gbuf, zbuf, s_sh,
                       sem0, sem1):
    cid = lax.axis_index("c")
    sid = lax.axis_index("s")
    wid = cid * _NS + sid
    base = wid * _EPT
    pltpu.sync_copy(send.at[pl.ds(base, _EPT)], sidx)
    pltpu.sync_copy(recv.at[pl.ds(base, _EPT)], ridx)
    _zero_shared_rows(zbuf, s_sh, sid)
    plsc.subcore_barrier()

    sems = (sem0, sem1)

    def gather(j, b, sem):
        return pltpu.async_copy(
            hn.at[sidx.at[pl.ds(j * _CHUNK, _CHUNK)]], gbuf.at[b], sem)

    gather(0, 0, sem0)

    def outer(g, carry):
        for b in range(2):
            j = g * 2 + b
            nxt = j + 1

            @pl.when(nxt < _CPT)
            def _():
                gather(nxt, 1 - b, sems[1 - b])

            pltpu.make_async_copy(
                hn.at[sidx.at[pl.ds(j * _CHUNK, _CHUNK)]], gbuf.at[b],
                sems[b]).wait()
            pltpu.sync_copy(gbuf.at[b],
                            s_sh.at[ridx.at[pl.ds(j * _CHUNK, _CHUNK)]],
                            add=True)
        return carry
    lax.fori_loop(0, _CPT // 2, outer, 0)

    plsc.subcore_barrier()
    _drain_shared(s_sh, out, cid, sid)


@functools.partial(
    pl.kernel,
    mesh=_mesh,
    out_type=jax.ShapeDtypeStruct((_NC, _NP, _H), jnp.float32),
    scratch_types=[
        pltpu.VMEM((_EPT,), jnp.int32),
        pltpu.VMEM((2, _CHUNK, _H), jnp.float32),
        pltpu.VMEM((_CHUNK, _H), jnp.float32),
        pltpu.VMEM_SHARED((_NP, _H), jnp.float32),
        pltpu.SemaphoreType.DMA,
        pltpu.SemaphoreType.DMA,
    ],
    compiler_params=_sc_params,
)
def _sc_segment_sum(vals, recv, out, ridx, gbuf, zbuf, s_sh, sem0, sem1):
    cid = lax.axis_index("c")
    sid = lax.axis_index("s")
    wid = cid * _NS + sid
    base = wid * _EPT
    pltpu.sync_copy(recv.at[pl.ds(base, _EPT)], ridx)
    _zero_shared_rows(zbuf, s_sh, sid)
    plsc.subcore_barrier()

    sems = (sem0, sem1)

    def load(j, b, sem):
        return pltpu.async_copy(
            vals.at[pl.ds(base + j * _CHUNK, _CHUNK)], gbuf.at[b], sem)

    load(0, 0, sem0)

    def outer(g, carry):
        for b in range(2):
            j = g * 2 + b
            nxt = j + 1

            @pl.when(nxt < _CPT)
            def _():
                load(nxt, 1 - b, sems[1 - b])

            pltpu.make_async_copy(
                vals.at[pl.ds(base + j * _CHUNK, _CHUNK)], gbuf.at[b],
                sems[b]).wait()
            pltpu.sync_copy(gbuf.at[b],
                            s_sh.at[ridx.at[pl.ds(j * _CHUNK, _CHUNK)]],
                            add=True)
        return carry
    lax.fori_loop(0, _CPT // 2, outer, 0)

    plsc.subcore_barrier()
    _drain_shared(s_sh, out, cid, sid)


def _full(shape):
    return pl.BlockSpec(shape, lambda i: (0,) * len(shape))


def _mlp2_kernel(x_ref, w1_ref, b1_ref, w2_ref, b2_ref, o_ref):
    y = jnp.dot(x_ref[...], w1_ref[...], preferred_element_type=jnp.float32)
    y = jnp.maximum(y + b1_ref[...], 0.0)
    o_ref[...] = jnp.dot(y, w2_ref[...], preferred_element_type=jnp.float32) + b2_ref[...]


def _mlp2(x, w1, b1, w2, b2, rows_per_block, out_rows=None):
    rows, din = x.shape
    dmid = w1.shape[1]
    dout = w2.shape[1]
    out_rows = rows if out_rows is None else out_rows
    grid = out_rows // rows_per_block
    return pl.pallas_call(
        _mlp2_kernel,
        grid=(grid,),
        in_specs=[
            pl.BlockSpec((rows_per_block, din), lambda i: (i, 0)),
            _full((din, dmid)), _full((1, dmid)),
            _full((dmid, dout)), _full((1, dout)),
        ],
        out_specs=pl.BlockSpec((rows_per_block, dout), lambda i: (i, 0)),
        out_shape=jax.ShapeDtypeStruct((out_rows, dout), jnp.float32),
    )(x, w1, b1.reshape(1, -1), w2, b2.reshape(1, -1))


def _update_kernel(hn_ref, sp_ref0, sp_ref1, ae_ref0, ae_ref1,
                   wmt_ref, w0a_ref, w0b_ref, b0_ref, w1_ref, b1_ref,
                   wnode_ref, lns_ref, lnb_ref, o_ref):
    h = hn_ref[...]
    s = sp_ref0[0] + sp_ref1[0]
    agg = (jnp.dot(s, wmt_ref[...], preferred_element_type=jnp.float32)
           + ae_ref0[0] + ae_ref1[0])
    t = (jnp.dot(h, w0a_ref[...], preferred_element_type=jnp.float32)
         + jnp.dot(agg, w0b_ref[...], preferred_element_type=jnp.float32)
         + b0_ref[...])
    t = jnp.maximum(t, 0.0)
    no = jnp.dot(t, w1_ref[...], preferred_element_type=jnp.float32) + b1_ref[...]
    r = jnp.dot(h, wnode_ref[...], preferred_element_type=jnp.float32) + no
    mu = jnp.mean(r, axis=-1, keepdims=True)
    var = jnp.mean((r - mu) * (r - mu), axis=-1, keepdims=True)
    o_ref[...] = (r - mu) * lax.rsqrt(var + 1e-6) * lns_ref[...] + lnb_ref[...]


def _update(hn, s_p, agge_p, wmt, w0a, w0b, b0, w1, b1, wnode, lns, lnb):
    rb = 1024
    grid = _NP // rb
    row = lambda i: (i, 0)
    return pl.pallas_call(
        _update_kernel,
        grid=(grid,),
        in_specs=[
            pl.BlockSpec((rb, _H), row),
            pl.BlockSpec((1, rb, _H), lambda i: (0, i, 0)),
            pl.BlockSpec((1, rb, _H), lambda i: (1, i, 0)),
            pl.BlockSpec((1, rb, _H), lambda i: (0, i, 0)),
            pl.BlockSpec((1, rb, _H), lambda i: (1, i, 0)),
            _full((_H, _H)), _full((_H, _H)), _full((_H, _H)), _full((1, _H)),
            _full((_H, _H)), _full((1, _H)), _full((_H, _H)),
            _full((1, _H)), _full((1, _H)),
        ],
        out_specs=pl.BlockSpec((rb, _H), row),
        out_shape=jax.ShapeDtypeStruct((_NP, _H), jnp.float32),
    )(hn, s_p, s_p, agge_p, agge_p, wmt, w0a, w0b, b0.reshape(1, -1), w1,
      b1.reshape(1, -1), wnode, lns.reshape(1, -1), lnb.reshape(1, -1))


def kernel(nodes, edges, senders, receivers,
           enc_node_W0, enc_node_b0, enc_node_W1, enc_node_b1,
           enc_edge_W0, enc_edge_b0, enc_edge_W1, enc_edge_b1,
           W_message, W_node,
           nodeMLP_W0, nodeMLP_b0, nodeMLP_W1, nodeMLP_b1,
           ln_scale, ln_bias,
           dec_W0, dec_b0, dec_W1, dec_b1):
    f32 = jnp.float32

    pad = _EPAD - _E
    senders = senders.astype(jnp.int32)
    receivers = receivers.astype(jnp.int32)
    ar = jnp.arange(pad, dtype=jnp.int32)
    pad_send = (ar * 37) % _N
    pad_recv = _N + ar % (_NP - _N)
    send_p = jnp.concatenate([senders, pad_send])
    recv_p = jnp.concatenate([receivers, pad_recv])

    edges_r = edges.reshape(_E // 8, 32)
    edges_r = jnp.pad(edges_r, ((0, (_EPAD - _E) // 8), (0, 0)))
    nodes_p = jnp.pad(nodes, ((0, _NP - _N), (0, 0)))

    wm_top = W_message[:_H]
    wm_bot = W_message[_H:]
    eye8 = jnp.eye(8, dtype=f32)
    w2c = enc_edge_W1 @ wm_bot
    b2c = enc_edge_b1 @ wm_bot
    w1k = jnp.kron(eye8, enc_edge_W0)
    b1k = jnp.tile(enc_edge_b0, 8)
    w2k = jnp.kron(eye8, w2c)
    b2k = jnp.tile(b2c, 8)

    h_n = _mlp2(nodes_p, enc_node_W0, enc_node_b0, enc_node_W1, enc_node_b1, 1024)
    z_e = _mlp2(edges_r, w1k, b1k, w2k, b2k, 4096).reshape(_EPAD, _H)

    agge_p = _sc_segment_sum(z_e, recv_p)

    w0a = nodeMLP_W0[:_H]
    w0b = nodeMLP_W0[_H:]

    for _ in range(_NMP):
        s_p = _sc_gather_scatter(h_n, send_p, recv_p)
        h_n = _update(h_n, s_p, agge_p,
                      wm_top, w0a, w0b, nodeMLP_b0, nodeMLP_W1, nodeMLP_b1,
                      W_node, ln_scale, ln_bias)

    return _mlp2(h_n, dec_W0, dec_b0, dec_W1, dec_b1, 1000, out_rows=_N)
